# baseline (speedup 1.0000x reference)
import functools

import jax
import jax.numpy as jnp
from jax import lax
from jax.experimental import pallas as pl
from jax.experimental.pallas import tpu as pltpu
from jax.experimental.pallas import tpu_sc as plsc

N = 10000
MSG = 30
SDIM = 4
K = 8
TB = 128
NTB = (N + TB - 1) // TB
SRCW = SDIM + MSG
BIGI = 2 ** 30

NW = 32
CH = 320
NP = NW * CH
GC = 64


def _conv_body(src_ref, sT_ref, A_ref, idx_ref, ew_ref, d2_ref):
    d2 = (src_ref[:, 0:1] - sT_ref[0:1, :]) ** 2
    for c in range(1, SDIM):
        d2 = d2 + (src_ref[:, c:c + 1] - sT_ref[c:c + 1, :]) ** 2
    d2_ref[...] = d2
    riota = jax.lax.broadcasted_iota(jnp.int32, (N, TB), 0)
    jms, ews = [], []
    for k in range(K):
        d2v = d2_ref[...]
        m = jnp.min(d2v, axis=0, keepdims=True)
        jm = jnp.min(jnp.where(d2v == m, riota, BIGI), axis=0,
                     keepdims=True)
        ew = jnp.exp(-10.0 * m)
        idx_ref[k:k + 1, :] = jm
        ew_ref[k:k + 1, :] = ew
        jms.append(jm)
        ews.append(ew)
        if k < K - 1:
            d2_ref[...] = jnp.where(riota == jm, jnp.float32(jnp.inf), d2v)
    acc = jnp.where(riota == jms[0], ews[0], 0.0)
    for k in range(1, K):
        acc = acc + jnp.where(riota == jms[k], ews[k], 0.0)
    A_ref[...] = acc


def _conv(src, sT):
    return pl.pallas_call(
        _conv_body, grid=(NTB,),
        in_specs=[pl.BlockSpec((N, SRCW), lambda i: (0, 0)),
                  pl.BlockSpec((SDIM, TB), lambda i: (0, i))],
        out_specs=[pl.BlockSpec((N, TB), lambda i: (0, i)),
                   pl.BlockSpec((K, TB), lambda i: (0, i)),
                   pl.BlockSpec((K, TB), lambda i: (0, i))],
        out_shape=[jax.ShapeDtypeStruct((N, N), jnp.float32),
                   jax.ShapeDtypeStruct((K, N), jnp.int32),
                   jax.ShapeDtypeStruct((K, N), jnp.float32)],
        scratch_shapes=[pltpu.VMEM((N, TB), jnp.float32)],
    )(src, sT)


_sc_mesh = plsc.VectorSubcoreMesh(core_axis_name="c", subcore_axis_name="s")


@functools.partial(
    pl.kernel, mesh=_sc_mesh,
    out_type=jax.ShapeDtypeStruct((NP, 32), jnp.float32),
    scratch_types=[
        pltpu.VMEM((K * CH,), jnp.int32),
        pltpu.VMEM((K * CH,), jnp.float32),
        pltpu.VMEM((CH, 128), jnp.float32),
        pltpu.VMEM((CH, 32), jnp.float32),
        pltpu.SemaphoreType.DMA,
    ],
)
def _sc_agg(msgp_hbm, idxF_hbm, ewF_hbm, out_hbm, idx_v, ew_v, rows_v, acc_v,
            sem):
    wid = lax.axis_index("s") * 2 + lax.axis_index("c")
    base = wid * CH
    for k in range(K):
        pltpu.sync_copy(idxF_hbm.at[pl.ds(k * NP + base, CH)],
                        idx_v.at[pl.ds(k * CH, CH)])
        pltpu.sync_copy(ewF_hbm.at[pl.ds(k * NP + base, CH)],
                        ew_v.at[pl.ds(k * CH, CH)])
    for k in range(K):
        for g in range(0, CH, GC):
            pltpu.async_copy(
                msgp_hbm.at[idx_v.at[pl.ds(k * CH + g, GC)]],
                rows_v.at[pl.ds(g, GC)], sem).wait()

        def body(gi, carry, k=k):
            g = gi * 16
            ew_vec = ew_v[pl.ds(k * CH + g, 16)]
            for j in range(16):
                t = g + j
                w = ew_vec[j]
                r0 = rows_v[t, pl.ds(0, 16)]
                r1 = rows_v[t, pl.ds(16, 16)]
                p0 = w * r0
                p1 = w * r1
                if k == 0:
                    acc_v[t, pl.ds(0, 16)] = p0
                    acc_v[t, pl.ds(16, 16)] = p1
                else:
                    a0 = acc_v[t, pl.ds(0, 16)]
                    a1 = acc_v[t, pl.ds(16, 16)]
                    acc_v[t, pl.ds(0, 16)] = a0 + p0
                    acc_v[t, pl.ds(16, 16)] = a1 + p1
            return carry

        lax.fori_loop(0, CH // 16, body, 0)
    pltpu.sync_copy(acc_v, out_hbm.at[pl.ds(base, CH)])


def _mlp(h, layers):
    for i, l in enumerate(layers):
        h = h @ l["w"].T + l["b"]
        if i < len(layers) - 1:
            h = jax.nn.elu(h)
    return h


def kernel(x, params):
    emb = _mlp(x, params["nn1"])
    As, msgs = [], []
    for p in params["convs"]:
        msg = emb @ p["lin_p"]["w"].T + p["lin_p"]["b"]
        s = emb @ p["lin_s"]["w"].T + p["lin_s"]["b"]
        src = jnp.concatenate([s, msg], axis=1)
        A, idxT, ewT = _conv(src, s.T)
        idxF = jnp.pad(idxT, ((0, 0), (0, NP - N))).reshape(-1)
        ewF = jnp.pad(ewT, ((0, 0), (0, NP - N))).reshape(-1)
        msgp = jnp.pad(msg, ((0, 0), (0, 128 - MSG)))
        agg = _sc_agg(msgp, idxF, ewF)[:N, :MSG]
        emb = agg @ p["lin_out"]["w"].T + p["lin_out"]["b"]
        As.append(A)
        msgs.append(msg)
    preds_id = _mlp(jnp.concatenate([x, emb], axis=-1), params["nn2"])
    preds_p4 = _mlp(jnp.concatenate([x, preds_id], axis=-1), params["nn3"])
    preds = jnp.concatenate([preds_id, preds_p4], axis=-1)
    return (preds, As[0], As[1], msgs[0], msgs[1])


# speedup vs baseline: 1.0779x; 1.0779x over previous
import functools

import jax
import jax.numpy as jnp
from jax import lax
from jax.experimental import pallas as pl
from jax.experimental.pallas import tpu as pltpu
from jax.experimental.pallas import tpu_sc as plsc

N = 10000
MSG = 30
SDIM = 4
K = 8
TB = 256
NTB = (N + TB - 1) // TB
SRCW = SDIM + MSG
BIGI = 2 ** 30

NW = 32
CH = 320
NP = NW * CH
GC = 64


def _conv_body(src_ref, sT_ref, A_ref, idx_ref, ew_ref, d2_ref):
    d2 = (src_ref[:, 0:1] - sT_ref[0:1, :]) ** 2
    for c in range(1, SDIM):
        d2 = d2 + (src_ref[:, c:c + 1] - sT_ref[c:c + 1, :]) ** 2
    d2_ref[...] = d2
    A_ref[...] = jnp.zeros((N, TB), jnp.float32)
    riota = jax.lax.broadcasted_iota(jnp.int32, (N, TB), 0)
    for k in range(K):
        d2v = d2_ref[...]
        m = jnp.min(d2v, axis=0, keepdims=True)
        jm = jnp.min(jnp.where(d2v == m, riota, BIGI), axis=0,
                     keepdims=True)
        ew = jnp.exp(-10.0 * m)
        hit = riota == jm
        A_ref[...] = A_ref[...] + jnp.where(hit, ew, 0.0)
        d2_ref[...] = jnp.where(hit, jnp.float32(jnp.inf), d2v)
        idx_ref[k:k + 1, :] = jm
        ew_ref[k:k + 1, :] = ew


def _conv(src, sT):
    return pl.pallas_call(
        _conv_body, grid=(NTB,),
        in_specs=[pl.BlockSpec((N, SRCW), lambda i: (0, 0)),
                  pl.BlockSpec((SDIM, TB), lambda i: (0, i))],
        out_specs=[pl.BlockSpec((N, TB), lambda i: (0, i)),
                   pl.BlockSpec((K, TB), lambda i: (0, i)),
                   pl.BlockSpec((K, TB), lambda i: (0, i))],
        out_shape=[jax.ShapeDtypeStruct((N, N), jnp.float32),
                   jax.ShapeDtypeStruct((K, N), jnp.int32),
                   jax.ShapeDtypeStruct((K, N), jnp.float32)],
        scratch_shapes=[pltpu.VMEM((N, TB), jnp.float32)],
    )(src, sT)


_sc_mesh = plsc.VectorSubcoreMesh(core_axis_name="c", subcore_axis_name="s")


@functools.partial(
    pl.kernel, mesh=_sc_mesh,
    out_type=jax.ShapeDtypeStruct((NP, 32), jnp.float32),
    scratch_types=[
        pltpu.VMEM((K * CH,), jnp.int32),
        pltpu.VMEM((K * CH,), jnp.float32),
        pltpu.VMEM((CH, 128), jnp.float32),
        pltpu.VMEM((CH, 32), jnp.float32),
        pltpu.SemaphoreType.DMA,
    ],
)
def _sc_agg(msgp_hbm, idxF_hbm, ewF_hbm, out_hbm, idx_v, ew_v, rows_v, acc_v,
            sem):
    wid = lax.axis_index("s") * 2 + lax.axis_index("c")
    base = wid * CH
    for k in range(K):
        pltpu.sync_copy(idxF_hbm.at[pl.ds(k * NP + base, CH)],
                        idx_v.at[pl.ds(k * CH, CH)])
        pltpu.sync_copy(ewF_hbm.at[pl.ds(k * NP + base, CH)],
                        ew_v.at[pl.ds(k * CH, CH)])
    for k in range(K):
        for g in range(0, CH, GC):
            pltpu.async_copy(
                msgp_hbm.at[idx_v.at[pl.ds(k * CH + g, GC)]],
                rows_v.at[pl.ds(g, GC)], sem).wait()

        def body(gi, carry, k=k):
            g = gi * 16
            ew_vec = ew_v[pl.ds(k * CH + g, 16)]
            for j in range(16):
                t = g + j
                w = ew_vec[j]
                r0 = rows_v[t, pl.ds(0, 16)]
                r1 = rows_v[t, pl.ds(16, 16)]
                p0 = w * r0
                p1 = w * r1
                if k == 0:
                    acc_v[t, pl.ds(0, 16)] = p0
                    acc_v[t, pl.ds(16, 16)] = p1
                else:
                    a0 = acc_v[t, pl.ds(0, 16)]
                    a1 = acc_v[t, pl.ds(16, 16)]
                    acc_v[t, pl.ds(0, 16)] = a0 + p0
                    acc_v[t, pl.ds(16, 16)] = a1 + p1
            return carry

        lax.fori_loop(0, CH // 16, body, 0)
    pltpu.sync_copy(acc_v, out_hbm.at[pl.ds(base, CH)])


def _mlp(h, layers):
    for i, l in enumerate(layers):
        h = h @ l["w"].T + l["b"]
        if i < len(layers) - 1:
            h = jax.nn.elu(h)
    return h


def kernel(x, params):
    emb = _mlp(x, params["nn1"])
    As, msgs = [], []
    for p in params["convs"]:
        msg = emb @ p["lin_p"]["w"].T + p["lin_p"]["b"]
        s = emb @ p["lin_s"]["w"].T + p["lin_s"]["b"]
        src = jnp.concatenate([s, msg], axis=1)
        A, idxT, ewT = _conv(src, s.T)
        idxF = jnp.pad(idxT, ((0, 0), (0, NP - N))).reshape(-1)
        ewF = jnp.pad(ewT, ((0, 0), (0, NP - N))).reshape(-1)
        msgp = jnp.pad(msg, ((0, 0), (0, 128 - MSG)))
        agg = _sc_agg(msgp, idxF, ewF)[:N, :MSG]
        emb = agg @ p["lin_out"]["w"].T + p["lin_out"]["b"]
        As.append(A)
        msgs.append(msg)
    preds_id = _mlp(jnp.concatenate([x, emb], axis=-1), params["nn2"])
    preds_p4 = _mlp(jnp.concatenate([x, preds_id], axis=-1), params["nn3"])
    preds = jnp.concatenate([preds_id, preds_p4], axis=-1)
    return (preds, As[0], As[1], msgs[0], msgs[1])
